# speedup vs baseline: 6.4556x; 6.4556x over previous
import functools

import jax
import jax.numpy as jnp
import numpy as np
from jax.experimental import pallas as pl
from jax.experimental.pallas import tpu as pltpu

DM = 64
DI = 128
NS = 16
RK = 4
H = 64
W = 64
L = H * W
T = 32
G = L // T

_F32 = jnp.float32


def _conv2d(x, w, b=None, pad=0, dil=1, groups=1):
    y = jax.lax.conv_general_dilated(
        x, w, (1, 1), [(pad, pad), (pad, pad)], rhs_dilation=(dil, dil),
        feature_group_count=groups, dimension_numbers=('NCHW', 'OIHW', 'NCHW'))
    return y if b is None else y + b[None, :, None, None]


def _lrelu(x, s=0.01):
    return jnp.where(x >= 0, x, s * x)


def _ln2d(x, g, b):
    mu = x.mean(1, keepdims=True)
    v = ((x - mu) ** 2).mean(1, keepdims=True)
    return (x - mu) * jax.lax.rsqrt(v + 1e-6) * g[None, :, None, None] + b[None, :, None, None]


def _satt(a, b, w1, b1, w2, b2):
    f = jnp.concatenate([a, b], 1)
    return jax.nn.sigmoid(_conv2d(_lrelu(_conv2d(f, w1, b1, pad=1)), w2, b2, pad=1))


def _align(f1, f2, f3, p):
    a12 = _satt(f1, f2, p['a1w1'], p['a1b1'], p['a1w2'], p['a1b2'])
    a32 = _satt(f3, f2, p['a2w1'], p['a2b1'], p['a2w2'], p['a2b2'])
    return _conv2d(jnp.concatenate([f1 * a12, f2, f3 * a32], 1), p['ow'], p['ob'])


def _dilated(x, p):
    t = _conv2d(x, p['w1'], p['b1'])
    t = _lrelu(_conv2d(t, p['w2'], p['b2'], pad=2, dil=2, groups=DM // 4), 0.2)
    t = _lrelu(_conv2d(t, p['w3'], p['b3']), 0.2)
    t = _conv2d(t, p['w4'], p['b4'])
    return _conv2d(t, p['w5'], p['b5'], pad=2, dil=2, groups=DM)


def _postproc(x, p):
    inp = x
    x = _ln2d(x, p['ln1_g'], p['ln1_b'])
    x = _conv2d(_conv2d(x, p['s1w1'], p['s1b1']), p['s1w2'], p['s1b2'], pad=1, groups=DM)
    x1, x2 = jnp.split(x, 2, axis=1)
    x = x1 * x2
    x = _conv2d(x.mean((2, 3), keepdims=True), p['sca_w'], p['sca_b']) * x
    y = _conv2d(x, p['mid_w'], p['mid_b']) + inp
    x = _ln2d(y, p['ln2_g'], p['ln2_b'])
    x1, x2 = jnp.split(_conv2d(x, p['c1w'], p['c1b']), 2, axis=1)
    return _conv2d(x1 * x2, p['c2w'], p['c2b']) + y


def _softplus(x):
    return jnp.maximum(x, 0.0) + jnp.log1p(jnp.exp(-jnp.abs(x)))


def _silu(x):
    return x * jax.nn.sigmoid(x)


def _ln_last(x, g, b, eps):
    mu = jnp.mean(x, -1, keepdims=True)
    v = jnp.mean((x - mu) ** 2, -1, keepdims=True)
    return (x - mu) * jax.lax.rsqrt(v + eps) * g + b


def _block_kernel(x_ref, lng, lnb, in_w, convw, convb, aflat, dtproj, dtb,
                  wb, wc, dvec, kb, tril, triu, ong, onb, outw, s1,
                  fg, fb, fc1w, fc1b, fc2w, fc2b, s2,
                  out_ref,
                  acc3, xchw, xcwh, dts, bs, cs, yhw, ywh):
    x = x_ref[0]
    xn = _ln_last(x, lng[...], lnb[...], 1e-6)
    xz = jnp.dot(xn, in_w[...], preferred_element_type=_F32)
    xi = xz[:, :DI]
    z = xz[:, DI:]

    x3 = xi.reshape(H, W, DI)
    wcv = convw[...]
    acc3[...] = x3 * wcv[4].reshape(1, 1, DI)
    for di in (-1, 0, 1):
        for dj in (-1, 0, 1):
            if di == 0 and dj == 0:
                continue
            tap = wcv[(di + 1) * 3 + (dj + 1)].reshape(1, 1, DI)
            ha, hb = max(0, -di), H - max(0, di)
            wa, wb_ = max(0, -dj), W - max(0, dj)
            acc3[ha:hb, wa:wb_, :] = (acc3[ha:hb, wa:wb_, :]
                                      + x3[ha + di:hb + di, wa + dj:wb_ + dj, :] * tap)
    xc3 = _silu(acc3[...] + convb[...].reshape(1, 1, DI))
    xchw[...] = xc3.reshape(L, DI)
    xcwh[...] = jnp.transpose(xc3, (1, 0, 2)).reshape(L, DI)

    kbv = kb[...]
    trilv = tril[...]
    triuv = triu[...]
    dv = dvec[...]

    yhw[...] = xchw[...] * (dv[0] + dv[2])
    ywh[...] = xcwh[...] * (dv[1] + dv[3])

    for d in range(4):
        seq_ref = xchw if d in (0, 2) else xcwh
        tgt_ref = yhw if d in (0, 2) else ywh
        rev = d >= 2
        seq = seq_ref[...]
        dts[...] = _softplus(jnp.dot(seq, dtproj[d], preferred_element_type=_F32)
                             + dtb[d])
        bs[...] = jnp.dot(seq, wb[d], preferred_element_type=_F32)
        cs[...] = jnp.dot(seq, wc[d], preferred_element_type=_F32)
        tri = triuv if rev else trilv
        af = aflat[d]

        def body(i, h0, *, rev=rev, tri=tri, af=af, seq_ref=seq_ref, tgt_ref=tgt_ref):
            gi = (G - 1 - i) if rev else i
            r0 = gi * T
            u = seq_ref[pl.ds(r0, T), :]
            dtc = dts[pl.ds(r0, T), :]
            Bc = bs[pl.ds(r0, T), :]
            Cc = cs[pl.ds(r0, T), :]
            dtA = pltpu.repeat(dtc, NS, axis=1) * af
            S = jnp.dot(tri, dtA, preferred_element_type=_F32)
            E = jnp.exp(S)
            wt = dtc * u
            Q = pltpu.repeat(wt, NS, axis=1) \
                * jnp.dot(Bc, kbv, preferred_element_type=_F32) / E
            R = jnp.dot(tri, Q, preferred_element_type=_F32)
            hf = E * (h0 + R)
            y = jnp.dot(Cc, kbv, preferred_element_type=_F32) * hf
            y = y[:, :8 * DI] + y[:, 8 * DI:]
            y = y[:, :4 * DI] + y[:, 4 * DI:]
            y = y[:, :2 * DI] + y[:, 2 * DI:]
            y = y[:, :DI] + y[:, DI:]
            tgt_ref[pl.ds(r0, T), :] = tgt_ref[pl.ds(r0, T), :] + y
            return hf[0:1, :] if rev else hf[T - 1:T, :]

        h0 = jnp.zeros((1, NS * DI), _F32)
        jax.lax.fori_loop(0, G, body, h0)

    ytot = yhw[...] + jnp.transpose(ywh[...].reshape(W, H, DI), (1, 0, 2)).reshape(L, DI)
    ytot = _ln_last(ytot, ong[...], onb[...], 1e-5)
    ytot = ytot * _silu(z)
    ss_out = jnp.dot(ytot, outw[...], preferred_element_type=_F32)
    x1 = ss_out + x * s1[...]
    t = _ln_last(x1, fg[...], fb[...], 1e-5)
    t = jnp.dot(_lrelu(jnp.dot(t, fc1w[...], preferred_element_type=_F32) + fc1b[...]),
                fc2w[...], preferred_element_type=_F32) + fc2b[...]
    out_ref[0] = t + x1 * s2[...]


def _prep_block(bp):
    ss = bp['ss']
    xp = ss['xproj_w']
    w_dt = jnp.transpose(xp[:, :RK, :], (0, 2, 1))
    wb = jnp.transpose(xp[:, RK:RK + NS, :], (0, 2, 1))
    wc = jnp.transpose(xp[:, RK + NS:, :], (0, 2, 1))
    dtw = ss['dt_w']
    dtproj = jnp.einsum('kdr,ker->kde', w_dt, dtw)
    a = -jnp.exp(ss['A_log'])
    aflat = jnp.transpose(a, (0, 2, 1)).reshape(4, 1, NS * DI)
    return (
        bp['ln_in_g'].reshape(1, DM), bp['ln_in_b'].reshape(1, DM),
        ss['in_w'],
        ss['conv_w'].reshape(DI, 9).T,
        ss['conv_b'].reshape(1, DI),
        aflat, dtproj, ss['dt_b'].reshape(4, 1, DI),
        wb, wc, ss['D'].reshape(4, 1, DI),
        ss['on_g'].reshape(1, DI), ss['on_b'].reshape(1, DI),
        ss['out_w'],
        bp['scale1'].reshape(1, DM),
        bp['ln_ffn_g'].reshape(1, DM), bp['ln_ffn_b'].reshape(1, DM),
        bp['fc1_w'], bp['fc1_b'].reshape(1, 2 * DM),
        bp['fc2_w'], bp['fc2_b'].reshape(1, DM),
        bp['scale2'].reshape(1, DM),
    )


_KB = None
_TRIL = None
_TRIU = None


def _consts():
    global _KB, _TRIL, _TRIU
    if _KB is None:
        kb = np.zeros((NS, NS * DI), np.float32)
        for n in range(NS):
            kb[n, n * DI:(n + 1) * DI] = 1.0
        _KB = jnp.asarray(kb)
        _TRIL = jnp.asarray(np.tril(np.ones((T, T), np.float32)))
        _TRIU = jnp.asarray(np.triu(np.ones((T, T), np.float32)))
    return _KB, _TRIL, _TRIU


def _block_call(xt, prep, interpret=False):
    kb, tril, triu = _consts()
    (lng, lnb, in_w, convw, convb, aflat, dtproj, dtb, wb, wc, dvec,
     ong, onb, outw, s1, fg, fb, fc1w, fc1b, fc2w, fc2b, s2) = prep
    b = xt.shape[0]
    weights = (lng, lnb, in_w, convw, convb, aflat, dtproj, dtb, wb, wc,
               dvec, kb, tril, triu, ong, onb, outw, s1, fg, fb,
               fc1w, fc1b, fc2w, fc2b, s2)
    in_specs = [pl.BlockSpec((1, L, DM), lambda i: (i, 0, 0))]
    for wgt in weights:
        nd = wgt.ndim
        in_specs.append(pl.BlockSpec(wgt.shape, (lambda i, nd=nd: (0,) * nd)))
    return pl.pallas_call(
        _block_kernel,
        out_shape=jax.ShapeDtypeStruct((b, L, DM), _F32),
        grid=(b,),
        in_specs=in_specs,
        out_specs=pl.BlockSpec((1, L, DM), lambda i: (i, 0, 0)),
        scratch_shapes=[
            pltpu.VMEM((H, W, DI), _F32),
            pltpu.VMEM((L, DI), _F32),
            pltpu.VMEM((L, DI), _F32),
            pltpu.VMEM((L, DI), _F32),
            pltpu.VMEM((L, NS), _F32),
            pltpu.VMEM((L, NS), _F32),
            pltpu.VMEM((L, DI), _F32),
            pltpu.VMEM((L, DI), _F32),
        ],
        compiler_params=pltpu.CompilerParams(
            dimension_semantics=("parallel",),
            vmem_limit_bytes=60 * 1024 * 1024,
        ),
        name="mamba_block",
        interpret=interpret,
    )(xt, *weights)


def _to_tok(x):
    return x.reshape(x.shape[0], DM, L).transpose(0, 2, 1)


def _from_tok(xt):
    return xt.transpose(0, 2, 1).reshape(xt.shape[0], DM, H, W)


@jax.jit
def _forward(f1, f2, f3, params):
    f1c = _conv2d(f1, params['f1w'], params['f1b'], pad=1)
    f2c = _conv2d(f2, params['f2w'], params['f2b'], pad=1)
    f3c = _conv2d(f3, params['f3w'], params['f3b'], pad=1)
    fused = _align(f1c, f2c, f3c, params['align'])
    x = fused
    for gp in params['groups']:
        gi = x
        xt = _to_tok(x)
        for bp in gp['blocks']:
            xt = _block_call(xt, _prep_block(bp))
        x = _dilated(_from_tok(xt), gp['dc']) + gi
    x = _conv2d(x + fused, params['cab_w'], params['cab_b'])
    x = _postproc(x, params['post'])
    x = _conv2d(x, params['oc_w1'], params['oc_b1'])
    x = _conv2d(x, params['oc_w2'], params['oc_b2'], pad=1, groups=DM)
    return jax.nn.sigmoid(_conv2d(x + f2c, params['al_w'], params['al_b'], pad=1))


def kernel(f1, f2, f3, params):
    return _forward(f1, f2, f3, params)


# speedup vs baseline: 7.8785x; 1.2204x over previous
import functools

import jax
import jax.numpy as jnp
import numpy as np
from jax.experimental import pallas as pl
from jax.experimental.pallas import tpu as pltpu

DM = 64
DI = 128
NS = 16
RK = 4
H = 64
W = 64
L = H * W
T = 32
G = L // T

_F32 = jnp.float32


def _conv2d(x, w, b=None, pad=0, dil=1, groups=1):
    y = jax.lax.conv_general_dilated(
        x, w, (1, 1), [(pad, pad), (pad, pad)], rhs_dilation=(dil, dil),
        feature_group_count=groups, dimension_numbers=('NCHW', 'OIHW', 'NCHW'))
    return y if b is None else y + b[None, :, None, None]


def _lrelu(x, s=0.01):
    return jnp.where(x >= 0, x, s * x)


def _ln2d(x, g, b):
    mu = x.mean(1, keepdims=True)
    v = ((x - mu) ** 2).mean(1, keepdims=True)
    return (x - mu) * jax.lax.rsqrt(v + 1e-6) * g[None, :, None, None] + b[None, :, None, None]


def _satt(a, b, w1, b1, w2, b2):
    f = jnp.concatenate([a, b], 1)
    return jax.nn.sigmoid(_conv2d(_lrelu(_conv2d(f, w1, b1, pad=1)), w2, b2, pad=1))


def _align(f1, f2, f3, p):
    a12 = _satt(f1, f2, p['a1w1'], p['a1b1'], p['a1w2'], p['a1b2'])
    a32 = _satt(f3, f2, p['a2w1'], p['a2b1'], p['a2w2'], p['a2b2'])
    return _conv2d(jnp.concatenate([f1 * a12, f2, f3 * a32], 1), p['ow'], p['ob'])


def _dilated(x, p):
    t = _conv2d(x, p['w1'], p['b1'])
    t = _lrelu(_conv2d(t, p['w2'], p['b2'], pad=2, dil=2, groups=DM // 4), 0.2)
    t = _lrelu(_conv2d(t, p['w3'], p['b3']), 0.2)
    t = _conv2d(t, p['w4'], p['b4'])
    return _conv2d(t, p['w5'], p['b5'], pad=2, dil=2, groups=DM)


def _postproc(x, p):
    inp = x
    x = _ln2d(x, p['ln1_g'], p['ln1_b'])
    x = _conv2d(_conv2d(x, p['s1w1'], p['s1b1']), p['s1w2'], p['s1b2'], pad=1, groups=DM)
    x1, x2 = jnp.split(x, 2, axis=1)
    x = x1 * x2
    x = _conv2d(x.mean((2, 3), keepdims=True), p['sca_w'], p['sca_b']) * x
    y = _conv2d(x, p['mid_w'], p['mid_b']) + inp
    x = _ln2d(y, p['ln2_g'], p['ln2_b'])
    x1, x2 = jnp.split(_conv2d(x, p['c1w'], p['c1b']), 2, axis=1)
    return _conv2d(x1 * x2, p['c2w'], p['c2b']) + y


def _softplus(x):
    return jnp.maximum(x, 0.0) + jnp.log1p(jnp.exp(-jnp.abs(x)))


def _silu(x):
    return x * jax.nn.sigmoid(x)


def _ln_last(x, g, b, eps):
    mu = jnp.mean(x, -1, keepdims=True)
    v = jnp.mean((x - mu) ** 2, -1, keepdims=True)
    return (x - mu) * jax.lax.rsqrt(v + eps) * g + b


def _block_kernel(x_ref, lng, lnb, in_w, convw, convb, aflat, dtproj, dtb,
                  wbc, dvec, kb, tril, triu, ong, onb, outw, s1,
                  fg, fb, fc1w, fc1b, fc2w, fc2b, s2,
                  out_ref,
                  acc3, xchw, xcwh, dts, bcs, yhw, ywh):
    x = x_ref[0]
    xn = _ln_last(x, lng[...], lnb[...], 1e-6)
    xz = jnp.dot(xn, in_w[...], preferred_element_type=_F32)
    xi = xz[:, :DI]
    z = xz[:, DI:]

    x3 = xi.reshape(H, W, DI)
    wcv = convw[...]
    acc3[...] = x3 * wcv[4].reshape(1, 1, DI)
    for di in (-1, 0, 1):
        for dj in (-1, 0, 1):
            if di == 0 and dj == 0:
                continue
            tap = wcv[(di + 1) * 3 + (dj + 1)].reshape(1, 1, DI)
            ha, hb = max(0, -di), H - max(0, di)
            wa, wb_ = max(0, -dj), W - max(0, dj)
            acc3[ha:hb, wa:wb_, :] = (acc3[ha:hb, wa:wb_, :]
                                      + x3[ha + di:hb + di, wa + dj:wb_ + dj, :] * tap)
    xc3 = _silu(acc3[...] + convb[...].reshape(1, 1, DI))
    xchw[...] = xc3.reshape(L, DI)
    xcwh[...] = jnp.transpose(xc3, (1, 0, 2)).reshape(L, DI)

    kbv = kb[...]
    trilv = tril[...]
    triuv = triu[...]
    dv = dvec[...]

    yhw[...] = xchw[...] * (dv[0] + dv[2])
    ywh[...] = xcwh[...] * (dv[1] + dv[3])

    for d in range(4):
        seq = xchw[...] if d in (0, 2) else xcwh[...]
        dts[d] = _softplus(jnp.dot(seq, dtproj[d], preferred_element_type=_F32)
                           + dtb[d])
        bcs[d] = jnp.dot(seq, wbc[d], preferred_element_type=_F32)

    seq_refs = (xchw, xcwh, xchw, xcwh)
    tgt_refs = (yhw, ywh, yhw, ywh)
    afv = aflat[...]

    def body(i, carry):
        out = []
        for d in range(4):
            h0 = carry[d]
            rev = d >= 2
            tri = triuv if rev else trilv
            gi = (G - 1 - i) if rev else i
            r0 = gi * T
            u = seq_refs[d][pl.ds(r0, T), :]
            dtc = dts.at[d][pl.ds(r0, T), :]
            Bc = bcs.at[d][pl.ds(r0, T), :NS]
            Cc = bcs.at[d][pl.ds(r0, T), NS:]
            dtA = pltpu.repeat(dtc, NS, axis=1) * afv[d]
            S = jnp.dot(tri, dtA, preferred_element_type=_F32)
            E = jnp.exp(S)
            wt = dtc * u
            Q = pltpu.repeat(wt, NS, axis=1) \
                * jnp.dot(Bc, kbv, preferred_element_type=_F32) / E
            R = jnp.dot(tri, Q, preferred_element_type=_F32)
            hf = E * (h0 + R)
            y = jnp.dot(Cc, kbv, preferred_element_type=_F32) * hf
            y = y[:, :8 * DI] + y[:, 8 * DI:]
            y = y[:, :4 * DI] + y[:, 4 * DI:]
            y = y[:, :2 * DI] + y[:, 2 * DI:]
            y = y[:, :DI] + y[:, DI:]
            tgt_refs[d][pl.ds(r0, T), :] = tgt_refs[d][pl.ds(r0, T), :] + y
            out.append(hf[0:1, :] if rev else hf[T - 1:T, :])
        return tuple(out)

    h0s = tuple(jnp.zeros((1, NS * DI), _F32) for _ in range(4))
    jax.lax.fori_loop(0, G, body, h0s)

    ytot = yhw[...] + jnp.transpose(ywh[...].reshape(W, H, DI), (1, 0, 2)).reshape(L, DI)
    ytot = _ln_last(ytot, ong[...], onb[...], 1e-5)
    ytot = ytot * _silu(z)
    ss_out = jnp.dot(ytot, outw[...], preferred_element_type=_F32)
    x1 = ss_out + x * s1[...]
    t = _ln_last(x1, fg[...], fb[...], 1e-5)
    t = jnp.dot(_lrelu(jnp.dot(t, fc1w[...], preferred_element_type=_F32) + fc1b[...]),
                fc2w[...], preferred_element_type=_F32) + fc2b[...]
    out_ref[0] = t + x1 * s2[...]


def _prep_block(bp):
    ss = bp['ss']
    xp = ss['xproj_w']
    w_dt = jnp.transpose(xp[:, :RK, :], (0, 2, 1))
    wb = jnp.transpose(xp[:, RK:RK + NS, :], (0, 2, 1))
    wc = jnp.transpose(xp[:, RK + NS:, :], (0, 2, 1))
    dtw = ss['dt_w']
    dtproj = jnp.einsum('kdr,ker->kde', w_dt, dtw)
    a = -jnp.exp(ss['A_log'])
    aflat = jnp.transpose(a, (0, 2, 1)).reshape(4, 1, NS * DI)
    return (
        bp['ln_in_g'].reshape(1, DM), bp['ln_in_b'].reshape(1, DM),
        ss['in_w'],
        ss['conv_w'].reshape(DI, 9).T,
        ss['conv_b'].reshape(1, DI),
        aflat, dtproj, ss['dt_b'].reshape(4, 1, DI),
        jnp.concatenate([wb, wc], axis=2), ss['D'].reshape(4, 1, DI),
        ss['on_g'].reshape(1, DI), ss['on_b'].reshape(1, DI),
        ss['out_w'],
        bp['scale1'].reshape(1, DM),
        bp['ln_ffn_g'].reshape(1, DM), bp['ln_ffn_b'].reshape(1, DM),
        bp['fc1_w'], bp['fc1_b'].reshape(1, 2 * DM),
        bp['fc2_w'], bp['fc2_b'].reshape(1, DM),
        bp['scale2'].reshape(1, DM),
    )


_KB = None
_TRIL = None
_TRIU = None


def _consts():
    global _KB, _TRIL, _TRIU
    if _KB is None:
        kb = np.zeros((NS, NS * DI), np.float32)
        for n in range(NS):
            kb[n, n * DI:(n + 1) * DI] = 1.0
        _KB = jnp.asarray(kb)
        _TRIL = jnp.asarray(np.tril(np.ones((T, T), np.float32)))
        _TRIU = jnp.asarray(np.triu(np.ones((T, T), np.float32)))
    return _KB, _TRIL, _TRIU


def _block_call(xt, prep, interpret=False):
    kb, tril, triu = _consts()
    (lng, lnb, in_w, convw, convb, aflat, dtproj, dtb, wbc, dvec,
     ong, onb, outw, s1, fg, fb, fc1w, fc1b, fc2w, fc2b, s2) = prep
    b = xt.shape[0]
    weights = (lng, lnb, in_w, convw, convb, aflat, dtproj, dtb, wbc,
               dvec, kb, tril, triu, ong, onb, outw, s1, fg, fb,
               fc1w, fc1b, fc2w, fc2b, s2)
    in_specs = [pl.BlockSpec((1, L, DM), lambda i: (i, 0, 0))]
    for wgt in weights:
        nd = wgt.ndim
        in_specs.append(pl.BlockSpec(wgt.shape, (lambda i, nd=nd: (0,) * nd)))
    return pl.pallas_call(
        _block_kernel,
        out_shape=jax.ShapeDtypeStruct((b, L, DM), _F32),
        grid=(b,),
        in_specs=in_specs,
        out_specs=pl.BlockSpec((1, L, DM), lambda i: (i, 0, 0)),
        scratch_shapes=[
            pltpu.VMEM((H, W, DI), _F32),
            pltpu.VMEM((L, DI), _F32),
            pltpu.VMEM((L, DI), _F32),
            pltpu.VMEM((4, L, DI), _F32),
            pltpu.VMEM((4, L, 2 * NS), _F32),
            pltpu.VMEM((L, DI), _F32),
            pltpu.VMEM((L, DI), _F32),
        ],
        compiler_params=pltpu.CompilerParams(
            dimension_semantics=("parallel",),
            vmem_limit_bytes=60 * 1024 * 1024,
        ),
        name="mamba_block",
        interpret=interpret,
    )(xt, *weights)


def _to_tok(x):
    return x.reshape(x.shape[0], DM, L).transpose(0, 2, 1)


def _from_tok(xt):
    return xt.transpose(0, 2, 1).reshape(xt.shape[0], DM, H, W)


@jax.jit
def _forward(f1, f2, f3, params):
    f1c = _conv2d(f1, params['f1w'], params['f1b'], pad=1)
    f2c = _conv2d(f2, params['f2w'], params['f2b'], pad=1)
    f3c = _conv2d(f3, params['f3w'], params['f3b'], pad=1)
    fused = _align(f1c, f2c, f3c, params['align'])
    x = fused
    for gp in params['groups']:
        gi = x
        xt = _to_tok(x)
        for bp in gp['blocks']:
            xt = _block_call(xt, _prep_block(bp))
        x = _dilated(_from_tok(xt), gp['dc']) + gi
    x = _conv2d(x + fused, params['cab_w'], params['cab_b'])
    x = _postproc(x, params['post'])
    x = _conv2d(x, params['oc_w1'], params['oc_b1'])
    x = _conv2d(x, params['oc_w2'], params['oc_b2'], pad=1, groups=DM)
    return jax.nn.sigmoid(_conv2d(x + f2c, params['al_w'], params['al_b'], pad=1))


def kernel(f1, f2, f3, params):
    return _forward(f1, f2, f3, params)
